# baseline (speedup 1.0000x reference)
import jax
import jax.numpy as jnp
from jax import lax
from jax.experimental import pallas as pl
from jax.experimental.pallas import tpu as pltpu, tpu_sc as plsc

BATCH = 4096
SEQ = 200
DIM = 64
NC, NS, L = 2, 16, 16
NW = NC * NS
ROWS_PER_W = BATCH // NW
TOK_PER_W = ROWS_PER_W * SEQ
GROUP_ROWS = 16
GROUP_TOK = GROUP_ROWS * SEQ
N_GROUPS = ROWS_PER_W // GROUP_ROWS
PHASE_SEQS = 2
PHASE_ROWS = PHASE_SEQS * SEQ
GATHER_SPLIT = (128, 128, 128, 16)
PHASES_PER_GROUP = GROUP_ROWS // PHASE_SEQS
N_PHASES = N_GROUPS * PHASES_PER_GROUP
TABLE_ROWS = 208


def _pos_embed_sc(tok_hbm, table_hbm, out_hbm, toks_v, idx_v, rows_v,
                  table_sp, tsem, gsem, wsem):
    wid = lax.axis_index("s") * NC + lax.axis_index("c")
    base = wid * TOK_PER_W
    seq_base = wid * ROWS_PER_W
    rowoff = lax.iota(jnp.int32, L) * SEQ
    ones = jnp.ones((L,), jnp.int32)
    zeros = jnp.zeros((L,), jnp.int32)

    @pl.when(lax.axis_index("s") == 0)
    def _():
        pltpu.sync_copy(table_hbm.at[pl.ds(0, TABLE_ROWS)], table_sp)

    plsc.subcore_barrier()

    def fire_tok(g):
        return pltpu.async_copy(
            tok_hbm.at[pl.ds(base + g * GROUP_TOK, GROUP_TOK)],
            toks_v.at[g % 2], tsem)

    def compute_positions(g):
        tv = toks_v.at[g % 2]
        iv = idx_v.at[g % 2]

        def pos_body(s, carry):
            idx = rowoff + s
            tok = plsc.load_gather(tv, [idx])
            m = jnp.where(tok != 0, ones, zeros)
            carry = carry + m
            plsc.store_scatter(iv, [idx], carry * m)
            return carry

        lax.fori_loop(0, SEQ, pos_body, zeros)

    def fire_gathers(ph):
        g, p = divmod(ph, PHASES_PER_GROUP)
        h = ph % 2
        descs = []
        o = 0
        for n in GATHER_SPLIT:
            descs.append(pltpu.async_copy(
                table_sp.at[idx_v.at[g % 2, pl.ds(p * PHASE_ROWS + o, n)]],
                rows_v.at[h, pl.ds(o, n)], gsem))
            o += n
        return descs

    def fire_writes(ph):
        g, p = divmod(ph, PHASES_PER_GROUP)
        h = ph % 2
        seq0 = seq_base + g * GROUP_ROWS + p * PHASE_SEQS
        return [
            pltpu.async_copy(
                rows_v.at[h, pl.ds(q * SEQ, SEQ)],
                out_hbm.at[seq0 + q], wsem)
            for q in range(PHASE_SEQS)
        ]

    gdescs, wdescs = {}, {}
    tok_desc = fire_tok(0)
    for g in range(N_GROUPS):
        tok_desc.wait()
        compute_positions(g)
        if g + 1 < N_GROUPS:
            tok_desc = fire_tok(g + 1)
        for p in range(PHASES_PER_GROUP):
            ph = g * PHASES_PER_GROUP + p
            if ph >= 1:
                for d in gdescs.pop(ph - 1):
                    d.wait()
                wdescs[ph - 1] = fire_writes(ph - 1)
            if ph >= 2:
                for d in wdescs.pop(ph - 2):
                    d.wait()
            gdescs[ph] = fire_gathers(ph)
    last = N_PHASES - 1
    for d in gdescs.pop(last):
        d.wait()
    wdescs[last] = fire_writes(last)
    for d in wdescs.pop(last - 1):
        d.wait()
    for d in wdescs.pop(last):
        d.wait()


@jax.jit
def kernel(input, embd_weights):
    tok_flat = input.reshape(-1).astype(jnp.int32)
    mesh = plsc.VectorSubcoreMesh(core_axis_name="c", subcore_axis_name="s")
    return pl.kernel(
        _pos_embed_sc,
        out_type=jax.ShapeDtypeStruct((BATCH, SEQ, DIM), jnp.float32),
        mesh=mesh,
        scratch_types=[
            pltpu.VMEM((2, GROUP_TOK), jnp.int32),
            pltpu.VMEM((2, GROUP_TOK), jnp.int32),
            pltpu.VMEM((2, PHASE_ROWS, DIM), jnp.float32),
            pltpu.VMEM_SHARED((TABLE_ROWS, DIM), jnp.float32),
            pltpu.SemaphoreType.DMA,
            pltpu.SemaphoreType.DMA,
            pltpu.SemaphoreType.DMA,
        ],
        compiler_params=pltpu.CompilerParams(
            needs_layout_passes=False, use_tc_tiling_on_sc=False
        ),
    )(tok_flat, embd_weights)


# speedup vs baseline: 1.3523x; 1.3523x over previous
import jax
import jax.numpy as jnp
from jax import lax
from jax.experimental import pallas as pl
from jax.experimental.pallas import tpu as pltpu, tpu_sc as plsc

BATCH = 4096
SEQ = 200
DIM = 64
NC, NS, L = 2, 16, 16
NW = NC * NS
ROWS_PER_W = BATCH // NW
TOK_PER_W = ROWS_PER_W * SEQ
GROUP_ROWS = 16
GROUP_TOK = GROUP_ROWS * SEQ
N_GROUPS = ROWS_PER_W // GROUP_ROWS
PHASE_SEQS = 2
PHASE_ROWS = PHASE_SEQS * SEQ
GATHER_SPLIT = (128, 128, 128, 16)
PHASES_PER_GROUP = GROUP_ROWS // PHASE_SEQS
N_PHASES = N_GROUPS * PHASES_PER_GROUP
TABLE_ROWS = 208


def _pos_embed_sc(tok_hbm, table_hbm, out_hbm,
                  toks0, toks1, idx0, idx1, rows0, rows1,
                  table_sp, tsem, gsem, wsem):
    wid = lax.axis_index("s") * NC + lax.axis_index("c")
    base = wid * TOK_PER_W
    seq_base = wid * ROWS_PER_W
    rowoff = lax.iota(jnp.int32, L) * SEQ
    ones = jnp.ones((L,), jnp.int32)
    zeros = jnp.zeros((L,), jnp.int32)
    toks = (toks0, toks1)
    idxs = (idx0, idx1)
    rows = (rows0, rows1)

    @pl.when(lax.axis_index("s") == 0)
    def _():
        pltpu.sync_copy(table_hbm.at[pl.ds(0, TABLE_ROWS)], table_sp)

    plsc.subcore_barrier()

    def fire_tok(g):
        return pltpu.async_copy(
            tok_hbm.at[pl.ds(base + g * GROUP_TOK, GROUP_TOK)],
            toks[g % 2], tsem)

    def compute_positions(g):
        tv = toks[g % 2]
        iv = idxs[g % 2]

        def pos_body(s, carry):
            idx = rowoff + s
            tok = plsc.load_gather(tv, [idx])
            m = jnp.where(tok != 0, ones, zeros)
            carry = carry + m
            plsc.store_scatter(iv, [idx], carry * m)
            return carry

        lax.fori_loop(0, SEQ, pos_body, zeros)

    def fire_gathers(ph):
        g, p = divmod(ph, PHASES_PER_GROUP)
        h = ph % 2
        descs = []
        o = 0
        for n in GATHER_SPLIT:
            descs.append(pltpu.async_copy(
                table_sp.at[idxs[g % 2].at[pl.ds(p * PHASE_ROWS + o, n)]],
                rows[h].at[pl.ds(o, n)], gsem))
            o += n
        return descs

    def fire_writes(ph):
        g, p = divmod(ph, PHASES_PER_GROUP)
        h = ph % 2
        seq0 = seq_base + g * GROUP_ROWS + p * PHASE_SEQS
        return [
            pltpu.async_copy(
                rows[h].at[pl.ds(q * SEQ, SEQ)],
                out_hbm.at[seq0 + q], wsem)
            for q in range(PHASE_SEQS)
        ]

    gdescs, wdescs = {}, {}
    tok_desc = fire_tok(0)
    for g in range(N_GROUPS):
        tok_desc.wait()
        compute_positions(g)
        if g + 1 < N_GROUPS:
            tok_desc = fire_tok(g + 1)
        for p in range(PHASES_PER_GROUP):
            ph = g * PHASES_PER_GROUP + p
            if ph >= 1:
                for d in gdescs.pop(ph - 1):
                    d.wait()
                wdescs[ph - 1] = fire_writes(ph - 1)
            if ph >= 2:
                for d in wdescs.pop(ph - 2):
                    d.wait()
            gdescs[ph] = fire_gathers(ph)
    last = N_PHASES - 1
    for d in gdescs.pop(last):
        d.wait()
    wdescs[last] = fire_writes(last)
    for d in wdescs.pop(last - 1):
        d.wait()
    for d in wdescs.pop(last):
        d.wait()


@jax.jit
def kernel(input, embd_weights):
    tok_flat = input.reshape(-1).astype(jnp.int32)
    mesh = plsc.VectorSubcoreMesh(core_axis_name="c", subcore_axis_name="s")
    return pl.kernel(
        _pos_embed_sc,
        out_type=jax.ShapeDtypeStruct((BATCH, SEQ, DIM), jnp.float32),
        mesh=mesh,
        scratch_types=[
            pltpu.VMEM((GROUP_TOK,), jnp.int32),
            pltpu.VMEM((GROUP_TOK,), jnp.int32),
            pltpu.VMEM((GROUP_TOK,), jnp.int32),
            pltpu.VMEM((GROUP_TOK,), jnp.int32),
            pltpu.VMEM((PHASE_ROWS, DIM), jnp.float32),
            pltpu.VMEM((PHASE_ROWS, DIM), jnp.float32),
            pltpu.VMEM_SHARED((TABLE_ROWS, DIM), jnp.float32),
            pltpu.SemaphoreType.DMA,
            pltpu.SemaphoreType.DMA,
            pltpu.SemaphoreType.DMA,
        ],
        compiler_params=pltpu.CompilerParams(
            needs_layout_passes=False, use_tc_tiling_on_sc=True
        ),
    )(tok_flat, embd_weights)
